# Initial kernel scaffold; baseline (speedup 1.0000x reference)
import jax
import jax.numpy as jnp
from jax.experimental import pallas as pl


def kernel(x, edge_index, batch, center_idx, params):
    raise NotImplementedError("write your pallas kernel here")


# speedup vs baseline: 17.4365x; 17.4365x over previous
import functools

import jax
import jax.numpy as jnp
from jax import lax
from jax.experimental import pallas as pl
from jax.experimental.pallas import tpu as pltpu
from jax.experimental.pallas import tpu_sc as plsc

N = 10000
E = 320000
DF = 128
H = 64
G = 16

NC = 2
NS = 16
NW = NC * NS
B = 128
NB_W = -(-E // (NW * B))
EPAD = NW * B * NB_W
NPAD = 10240
RPT = NPAD // NS
DW = 8


def _deg_body(dstr, zeros8, ones8, out, didx, ones_v, deg_sh):
    cid = lax.axis_index("c")
    sid = lax.axis_index("s")
    wid = cid * NS + sid
    pltpu.sync_copy(zeros8.at[pl.ds(sid * RPT, RPT)],
                    deg_sh.at[pl.ds(sid * RPT, RPT)])
    pltpu.sync_copy(ones8, ones_v)
    pltpu.sync_copy(dstr.at[wid], didx)
    plsc.subcore_barrier()
    for j in range(NB_W):
        pltpu.sync_copy(ones_v, deg_sh.at[didx.at[j]], add=True)
    plsc.subcore_barrier()
    pltpu.sync_copy(deg_sh.at[pl.ds(sid * RPT, RPT)],
                    out.at[cid, pl.ds(sid * RPT, RPT)])


@functools.cache
def _deg_call():
    mesh = plsc.VectorSubcoreMesh(
        core_axis_name="c", subcore_axis_name="s",
        num_cores=NC, num_subcores=NS)
    return pl.kernel(
        _deg_body,
        out_type=jax.ShapeDtypeStruct((NC, NPAD, DW), jnp.float32),
        mesh=mesh,
        scratch_types=[
            pltpu.VMEM((NB_W, B), jnp.int32),
            pltpu.VMEM((B, DW), jnp.float32),
            pltpu.VMEM_SHARED((NPAD, DW), jnp.float32),
        ],
        compiler_params=pltpu.CompilerParams(use_tc_tiling_on_sc=False),
    )


def _edge_body(hs, srcr, dstr, zeros_h, out,
               sidx, didx, rows0, rows1, rows2, acc_sh, s0, s1, s2):
    cid = lax.axis_index("c")
    sid = lax.axis_index("s")
    wid = cid * NS + sid
    pltpu.sync_copy(zeros_h.at[pl.ds(sid * RPT, RPT)],
                    acc_sh.at[pl.ds(sid * RPT, RPT)])
    pltpu.sync_copy(srcr.at[wid], sidx)
    pltpu.sync_copy(dstr.at[wid], didx)
    plsc.subcore_barrier()
    rows = (rows0, rows1, rows2)
    sems = (s0, s1, s2)
    descs = [None] * NB_W
    for j in range(min(2, NB_W)):
        descs[j] = pltpu.async_copy(hs.at[sidx.at[j]], rows[j % 3], sems[j % 3])
    for j in range(NB_W):
        if j + 2 < NB_W:
            descs[j + 2] = pltpu.async_copy(
                hs.at[sidx.at[j + 2]], rows[(j + 2) % 3], sems[(j + 2) % 3])
        descs[j].wait()
        pltpu.sync_copy(rows[j % 3], acc_sh.at[didx.at[j]], add=True)
    plsc.subcore_barrier()
    pltpu.sync_copy(acc_sh.at[pl.ds(sid * RPT, RPT)],
                    out.at[cid, pl.ds(sid * RPT, RPT)])


@functools.cache
def _edge_call():
    mesh = plsc.VectorSubcoreMesh(
        core_axis_name="c", subcore_axis_name="s",
        num_cores=NC, num_subcores=NS)
    return pl.kernel(
        _edge_body,
        out_type=jax.ShapeDtypeStruct((NC, NPAD, H), jnp.float32),
        mesh=mesh,
        scratch_types=[
            pltpu.VMEM((NB_W, B), jnp.int32),
            pltpu.VMEM((NB_W, B), jnp.int32),
            pltpu.VMEM((B, H), jnp.float32),
            pltpu.VMEM((B, H), jnp.float32),
            pltpu.VMEM((B, H), jnp.float32),
            pltpu.VMEM_SHARED((NPAD, H), jnp.float32),
            pltpu.SemaphoreType.DMA,
            pltpu.SemaphoreType.DMA,
            pltpu.SemaphoreType.DMA,
        ],
        compiler_params=pltpu.CompilerParams(use_tc_tiling_on_sc=False),
    )


BLK = 2000
NBLK = N // BLK


def _dot(a, b):
    return jnp.dot(a, b, preferred_element_type=jnp.float32,
                   precision=lax.Precision.HIGHEST)


def _segdot(pt, v):
    return lax.dot_general(pt, v, (((0,), (0,)), ((), ())),
                           preferred_element_type=jnp.float32,
                           precision=lax.Precision.HIGHEST)


def _pre_body(x_ref, w0_ref, d0_ref, d1_ref, dinv_ref, hs0_ref):
    d = d0_ref[...] + d1_ref[...] + 1.0
    dinv = lax.rsqrt(jnp.maximum(d, 1.0))
    dinv_ref[...] = dinv
    hs0_ref[...] = _dot(x_ref[...], w0_ref[...]) * dinv


_pre_call = pl.pallas_call(
    _pre_body,
    grid=(NBLK,),
    in_specs=[pl.BlockSpec((BLK, DF), lambda i: (i, 0)),
              pl.BlockSpec((DF, H), lambda i: (0, 0)),
              pl.BlockSpec((BLK, 1), lambda i: (i, 0)),
              pl.BlockSpec((BLK, 1), lambda i: (i, 0))],
    out_specs=[pl.BlockSpec((BLK, 1), lambda i: (i, 0)),
               pl.BlockSpec((BLK, H), lambda i: (i, 0))],
    out_shape=[jax.ShapeDtypeStruct((N, 1), jnp.float32),
               jax.ShapeDtypeStruct((N, H), jnp.float32)],
)


def _block_onehot(batch_blk):
    return (batch_blk == lax.broadcasted_iota(jnp.int32, (BLK, G), 1)
            ).astype(jnp.float32)


def _layer_body(with_res, with_mm, *refs):
    (acc0, acc1, hs_prev, dinv, batch, bvec, gw, gb, gms) = refs[:9]
    k = 9
    wn = refs[k] if with_mm else None
    k += int(with_mm)
    hres = refs[k] if with_res else None
    k += int(with_res)
    h_new_ref = refs[k]
    k += 1
    hs_next_ref = refs[k] if with_mm else None
    k += int(with_mm)
    cnt_s, sm_s, sq_s, mean_s, rstd_s = refs[k:k + 5]

    p = pl.program_id(0)
    i = pl.program_id(1)
    a = ((acc0[...] + acc1[...] + hs_prev[...]) * dinv[...]) + bvec[...]
    pt = _block_onehot(batch[...])

    @pl.when(jnp.logical_and(p == 0, i == 0))
    def _init():
        cnt_s[...] = jnp.zeros_like(cnt_s)
        sm_s[...] = jnp.zeros_like(sm_s)
        sq_s[...] = jnp.zeros_like(sq_s)

    @pl.when(p == 0)
    def _accum():
        ones = jnp.full((BLK, 1), 1.0, jnp.float32)
        cnt_s[...] += _segdot(pt, ones)
        sm_s[...] += _segdot(pt, a)
        sq_s[...] += _segdot(pt, a * a)

    @pl.when(jnp.logical_and(p == 1, i == 0))
    def _stats():
        c = jnp.maximum(cnt_s[...], 1.0)
        mean = sm_s[...] / c
        msv = gms[...]
        var = sq_s[...] / c - (2.0 * msv - msv * msv) * (mean * mean)
        mean_s[...] = mean
        rstd_s[...] = lax.rsqrt(var + 1e-5)

    @pl.when(p == 1)
    def _emit():
        ctr = a - _dot(pt, mean_s[...]) * gms[...]
        g = ctr * _dot(pt, rstd_s[...]) * gw[...] + gb[...]
        g = jnp.maximum(g, 0.0)
        h_new = (hres[...] + g) if with_res else g
        h_new_ref[...] = h_new
        if with_mm:
            hs_next_ref[...] = _dot(h_new, wn[...]) * dinv[...]


def _make_layer(with_res):
    row = lambda w: pl.BlockSpec((BLK, w), lambda p, i: (i, 0))
    const = lambda r, c: pl.BlockSpec((r, c), lambda p, i: (0, 0))
    in_specs = [row(H), row(H), row(H), row(1), row(1),
                const(1, H), const(1, H), const(1, H), const(1, H),
                const(H, H)]
    if with_res:
        in_specs.append(row(H))
    return pl.pallas_call(
        functools.partial(_layer_body, with_res, True),
        grid=(2, NBLK),
        in_specs=in_specs,
        out_specs=[row(H), row(H)],
        out_shape=[jax.ShapeDtypeStruct((N, H), jnp.float32),
                   jax.ShapeDtypeStruct((N, H), jnp.float32)],
        scratch_shapes=[pltpu.VMEM((G, 1), jnp.float32),
                        pltpu.VMEM((G, H), jnp.float32),
                        pltpu.VMEM((G, H), jnp.float32),
                        pltpu.VMEM((G, H), jnp.float32),
                        pltpu.VMEM((G, H), jnp.float32)],
    )


_mid_nores = _make_layer(False)
_mid_res = _make_layer(True)


def _final_body(acc0, acc1, hs3, dinv, batch, cent, bvec, gw, gb, gms,
                wm1, bm1, wm2, bm2, out_ref,
                cnt_s, sm_s, sq_s, mean_s, rstd_s, pool_s, cf_s):
    p = pl.program_id(0)
    i = pl.program_id(1)
    a = ((acc0[...] + acc1[...] + hs3[...]) * dinv[...]) + bvec[...]
    pt = _block_onehot(batch[...])

    @pl.when(jnp.logical_and(p == 0, i == 0))
    def _init():
        cnt_s[...] = jnp.zeros_like(cnt_s)
        sm_s[...] = jnp.zeros_like(sm_s)
        sq_s[...] = jnp.zeros_like(sq_s)
        pool_s[...] = jnp.zeros_like(pool_s)
        cf_s[...] = jnp.zeros_like(cf_s)

    @pl.when(p == 0)
    def _accum():
        ones = jnp.full((BLK, 1), 1.0, jnp.float32)
        cnt_s[...] += _segdot(pt, ones)
        sm_s[...] += _segdot(pt, a)
        sq_s[...] += _segdot(pt, a * a)

    @pl.when(jnp.logical_and(p == 1, i == 0))
    def _stats():
        c = jnp.maximum(cnt_s[...], 1.0)
        mean = sm_s[...] / c
        msv = gms[...]
        var = sq_s[...] / c - (2.0 * msv - msv * msv) * (mean * mean)
        mean_s[...] = mean
        rstd_s[...] = lax.rsqrt(var + 1e-5)

    @pl.when(p == 1)
    def _emit():
        ctr = a - _dot(pt, mean_s[...]) * gms[...]
        g = ctr * _dot(pt, rstd_s[...]) * gw[...] + gb[...]
        h = jnp.maximum(g, 0.0)
        pool_s[...] += _segdot(pt, h)
        rowid = lax.broadcasted_iota(jnp.int32, (BLK, G), 0) + i * BLK
        cm = (rowid == cent[...]).astype(jnp.float32)
        cf_s[...] += _segdot(cm, h)

    @pl.when(jnp.logical_and(p == 1, i == NBLK - 1))
    def _mlp():
        c = jnp.maximum(cnt_s[...], 1.0)
        xg = pool_s[...] / c
        xc = jnp.concatenate([xg, cf_s[...]], axis=1)
        m = jnp.maximum(_dot(xc, wm1[...]) + bm1[...], 0.0)
        out_ref[...] = _dot(m, wm2[...]) + bm2[...]


_final_call = pl.pallas_call(
    _final_body,
    grid=(2, NBLK),
    in_specs=[pl.BlockSpec((BLK, H), lambda p, i: (i, 0)),
              pl.BlockSpec((BLK, H), lambda p, i: (i, 0)),
              pl.BlockSpec((BLK, H), lambda p, i: (i, 0)),
              pl.BlockSpec((BLK, 1), lambda p, i: (i, 0)),
              pl.BlockSpec((BLK, 1), lambda p, i: (i, 0)),
              pl.BlockSpec((1, G), lambda p, i: (0, 0)),
              pl.BlockSpec((1, H), lambda p, i: (0, 0)),
              pl.BlockSpec((1, H), lambda p, i: (0, 0)),
              pl.BlockSpec((1, H), lambda p, i: (0, 0)),
              pl.BlockSpec((1, H), lambda p, i: (0, 0)),
              pl.BlockSpec((2 * H, H), lambda p, i: (0, 0)),
              pl.BlockSpec((1, H), lambda p, i: (0, 0)),
              pl.BlockSpec((H, 1), lambda p, i: (0, 0)),
              pl.BlockSpec((1, 1), lambda p, i: (0, 0))],
    out_specs=pl.BlockSpec((G, 1), lambda p, i: (0, 0)),
    out_shape=jax.ShapeDtypeStruct((G, 1), jnp.float32),
    scratch_shapes=[pltpu.VMEM((G, 1), jnp.float32),
                    pltpu.VMEM((G, H), jnp.float32),
                    pltpu.VMEM((G, H), jnp.float32),
                    pltpu.VMEM((G, H), jnp.float32),
                    pltpu.VMEM((G, H), jnp.float32),
                    pltpu.VMEM((G, H), jnp.float32),
                    pltpu.VMEM((G, H), jnp.float32)],
)


def kernel(x, edge_index, batch, center_idx, params):
    p = params
    src = edge_index[0].astype(jnp.int32)
    dst = edge_index[1].astype(jnp.int32)
    pad = EPAD - E
    srcr = jnp.concatenate([src, jnp.zeros((pad,), jnp.int32)]).reshape(NW, NB_W, B)
    dstr = jnp.concatenate([dst, jnp.full((pad,), N, jnp.int32)]).reshape(NW, NB_W, B)

    zeros8 = jnp.zeros((NPAD, DW), jnp.float32)
    ones8 = jnp.ones((B, DW), jnp.float32)
    zeros_h = jnp.zeros((NPAD, H), jnp.float32)

    degp = _deg_call()(dstr, zeros8, ones8)
    d0 = degp[0, :N, 0:1]
    d1 = degp[1, :N, 0:1]

    batch_col = batch.astype(jnp.int32)[:, None]
    cent_row = center_idx.astype(jnp.int32)[None, :]

    def vec(v):
        return v[None, :]

    dinv, hs = _pre_call(x, p['W0'], d0, d1)

    h_res = None
    for i in range(4):
        accp = _edge_call()(hs, srcr, dstr, zeros_h)
        acc0 = accp[0, :N, :]
        acc1 = accp[1, :N, :]
        if i < 3:
            args = (acc0, acc1, hs, dinv, batch_col, vec(p['b%d' % i]),
                    vec(p['gn%d_w' % i]), vec(p['gn%d_b' % i]),
                    vec(p['gn%d_ms' % i]), p['W%d' % (i + 1)])
            if i == 0:
                h_res, hs = _mid_nores(*args)
            else:
                h_res, hs = _mid_res(*args, h_res)
        else:
            out = _final_call(acc0, acc1, hs, dinv, batch_col, cent_row,
                              vec(p['b3']), vec(p['gn3_w']), vec(p['gn3_b']),
                              vec(p['gn3_ms']), p['Wm1'], vec(p['bm1']),
                              p['Wm2'], vec(p['bm2']))
    return out[:, 0]


# speedup vs baseline: 18.5388x; 1.0632x over previous
import functools

import jax
import jax.numpy as jnp
from jax import lax
from jax.experimental import pallas as pl
from jax.experimental.pallas import tpu as pltpu
from jax.experimental.pallas import tpu_sc as plsc

N = 10000
E = 320000
DF = 128
H = 64
G = 16

NC = 2
NS = 16
NW = NC * NS
B = 128
NB_W = -(-E // (NW * B))
EPAD = NW * B * NB_W
NPAD = 10240
RPT = NPAD // NS
DW = 8


def _deg_body(dstr, zeros8, ones8, out, didx, ones_v, deg_sh):
    cid = lax.axis_index("c")
    sid = lax.axis_index("s")
    wid = cid * NS + sid
    pltpu.sync_copy(zeros8.at[pl.ds(sid * RPT, RPT)],
                    deg_sh.at[pl.ds(sid * RPT, RPT)])
    pltpu.sync_copy(ones8, ones_v)
    pltpu.sync_copy(dstr.at[wid], didx)
    plsc.subcore_barrier()
    for j in range(NB_W):
        pltpu.sync_copy(ones_v, deg_sh.at[didx.at[j]], add=True)
    plsc.subcore_barrier()
    pltpu.sync_copy(deg_sh.at[pl.ds(sid * RPT, RPT)],
                    out.at[cid, pl.ds(sid * RPT, RPT)])


@functools.cache
def _deg_call():
    mesh = plsc.VectorSubcoreMesh(
        core_axis_name="c", subcore_axis_name="s",
        num_cores=NC, num_subcores=NS)
    return pl.kernel(
        _deg_body,
        out_type=jax.ShapeDtypeStruct((NC, NPAD, DW), jnp.float32),
        mesh=mesh,
        scratch_types=[
            pltpu.VMEM((NB_W, B), jnp.int32),
            pltpu.VMEM((B, DW), jnp.float32),
            pltpu.VMEM_SHARED((NPAD, DW), jnp.float32),
        ],
        compiler_params=pltpu.CompilerParams(use_tc_tiling_on_sc=False),
    )


def _edge_body(hs, srcr, dstr, zeros_h, out,
               sidx, didx, rows0, rows1, rows2, acc_sh, hs_sh, s0, s1, s2):
    cid = lax.axis_index("c")
    sid = lax.axis_index("s")
    wid = cid * NS + sid
    pltpu.sync_copy(zeros_h.at[pl.ds(sid * RPT, RPT)],
                    acc_sh.at[pl.ds(sid * RPT, RPT)])
    pltpu.sync_copy(hs.at[pl.ds(sid * RPT, RPT)],
                    hs_sh.at[pl.ds(sid * RPT, RPT)])
    pltpu.sync_copy(srcr.at[wid], sidx)
    pltpu.sync_copy(dstr.at[wid], didx)
    plsc.subcore_barrier()
    for j in range(NB_W):
        pltpu.sync_copy(hs_sh.at[sidx.at[j]], rows0)
        pltpu.sync_copy(rows0, acc_sh.at[didx.at[j]], add=True)
    plsc.subcore_barrier()
    pltpu.sync_copy(acc_sh.at[pl.ds(sid * RPT, RPT)],
                    out.at[cid, pl.ds(sid * RPT, RPT)])


@functools.cache
def _edge_call():
    mesh = plsc.VectorSubcoreMesh(
        core_axis_name="c", subcore_axis_name="s",
        num_cores=NC, num_subcores=NS)
    return pl.kernel(
        _edge_body,
        out_type=jax.ShapeDtypeStruct((NC, NPAD, H), jnp.float32),
        mesh=mesh,
        scratch_types=[
            pltpu.VMEM((NB_W, B), jnp.int32),
            pltpu.VMEM((NB_W, B), jnp.int32),
            pltpu.VMEM((B, H), jnp.float32),
            pltpu.VMEM((B, H), jnp.float32),
            pltpu.VMEM((B, H), jnp.float32),
            pltpu.VMEM_SHARED((NPAD, H), jnp.float32),
            pltpu.VMEM_SHARED((NPAD, H), jnp.float32),
            pltpu.SemaphoreType.DMA,
            pltpu.SemaphoreType.DMA,
            pltpu.SemaphoreType.DMA,
        ],
        compiler_params=pltpu.CompilerParams(use_tc_tiling_on_sc=False),
    )


BLK = 2000
NBLK = N // BLK


def _dot(a, b):
    return jnp.dot(a, b, preferred_element_type=jnp.float32,
                   precision=lax.Precision.HIGHEST)


def _dotd(a, b):
    return jnp.dot(a, b, preferred_element_type=jnp.float32)


def _segdot(pt, v):
    return lax.dot_general(pt, v, (((0,), (0,)), ((), ())),
                           preferred_element_type=jnp.float32,
                           precision=lax.Precision.HIGHEST)


def _pre_body(x_ref, w0_ref, d0_ref, d1_ref, dinv_ref, hs0_ref):
    d = d0_ref[...] + d1_ref[...] + 1.0
    dinv = lax.rsqrt(jnp.maximum(d, 1.0))
    dinv_ref[...] = dinv
    hs0_ref[...] = _dotd(x_ref[...], w0_ref[...]) * dinv


_pre_call = pl.pallas_call(
    _pre_body,
    grid=(NBLK,),
    in_specs=[pl.BlockSpec((BLK, DF), lambda i: (i, 0)),
              pl.BlockSpec((DF, H), lambda i: (0, 0)),
              pl.BlockSpec((BLK, 1), lambda i: (i, 0)),
              pl.BlockSpec((BLK, 1), lambda i: (i, 0))],
    out_specs=[pl.BlockSpec((BLK, 1), lambda i: (i, 0)),
               pl.BlockSpec((BLK, H), lambda i: (i, 0))],
    out_shape=[jax.ShapeDtypeStruct((N, 1), jnp.float32),
               jax.ShapeDtypeStruct((N, H), jnp.float32)],
)


def _block_onehot(batch_blk):
    return (batch_blk == lax.broadcasted_iota(jnp.int32, (BLK, G), 1)
            ).astype(jnp.float32)


def _layer_body(with_res, with_mm, *refs):
    (acc0, acc1, hs_prev, dinv, batch, bvec, gw, gb, gms) = refs[:9]
    k = 9
    wn = refs[k] if with_mm else None
    k += int(with_mm)
    hres = refs[k] if with_res else None
    k += int(with_res)
    h_new_ref = refs[k]
    k += 1
    hs_next_ref = refs[k] if with_mm else None
    k += int(with_mm)
    cnt_s, sm_s, sq_s, mean_s, rstd_s = refs[k:k + 5]

    p = pl.program_id(0)
    i = pl.program_id(1)
    a = ((acc0[...] + acc1[...] + hs_prev[...]) * dinv[...]) + bvec[...]
    pt = _block_onehot(batch[...])

    @pl.when(jnp.logical_and(p == 0, i == 0))
    def _init():
        cnt_s[...] = jnp.zeros_like(cnt_s)
        sm_s[...] = jnp.zeros_like(sm_s)
        sq_s[...] = jnp.zeros_like(sq_s)

    @pl.when(p == 0)
    def _accum():
        ones = jnp.full((BLK, 1), 1.0, jnp.float32)
        cnt_s[...] += _segdot(pt, ones)
        sm_s[...] += _segdot(pt, a)

    @pl.when(jnp.logical_and(p == 1, i == 0))
    def _mean():
        mean_s[...] = sm_s[...] / jnp.maximum(cnt_s[...], 1.0)

    @pl.when(p == 1)
    def _accum2():
        ctr = a - _dot(pt, mean_s[...]) * gms[...]
        sq_s[...] += _segdot(pt, ctr * ctr)

    @pl.when(jnp.logical_and(p == 2, i == 0))
    def _stats():
        c = jnp.maximum(cnt_s[...], 1.0)
        rstd_s[...] = jnp.sqrt(sq_s[...] / c + 1e-5)

    @pl.when(p == 2)
    def _emit():
        ctr = a - _dot(pt, mean_s[...]) * gms[...]
        g = ctr / _dot(pt, rstd_s[...]) * gw[...] + gb[...]
        g = jnp.maximum(g, 0.0)
        h_new = (hres[...] + g) if with_res else g
        h_new_ref[...] = h_new
        if with_mm:
            hs_next_ref[...] = _dotd(h_new, wn[...]) * dinv[...]


def _make_layer(with_res):
    row = lambda w: pl.BlockSpec((BLK, w), lambda p, i: (i, 0))
    const = lambda r, c: pl.BlockSpec((r, c), lambda p, i: (0, 0))
    in_specs = [row(H), row(H), row(H), row(1), row(1),
                const(1, H), const(1, H), const(1, H), const(1, H),
                const(H, H)]
    if with_res:
        in_specs.append(row(H))
    return pl.pallas_call(
        functools.partial(_layer_body, with_res, True),
        grid=(3, NBLK),
        in_specs=in_specs,
        out_specs=[row(H), row(H)],
        out_shape=[jax.ShapeDtypeStruct((N, H), jnp.float32),
                   jax.ShapeDtypeStruct((N, H), jnp.float32)],
        scratch_shapes=[pltpu.VMEM((G, 1), jnp.float32),
                        pltpu.VMEM((G, H), jnp.float32),
                        pltpu.VMEM((G, H), jnp.float32),
                        pltpu.VMEM((G, H), jnp.float32),
                        pltpu.VMEM((G, H), jnp.float32)],
    )


_mid_nores = _make_layer(False)
_mid_res = _make_layer(True)


def _final_body(acc0, acc1, hs3, dinv, batch, cent, bvec, gw, gb, gms,
                wm1, bm1, wm2, bm2, out_ref,
                cnt_s, sm_s, sq_s, mean_s, rstd_s, pool_s, cf_s):
    p = pl.program_id(0)
    i = pl.program_id(1)
    a = ((acc0[...] + acc1[...] + hs3[...]) * dinv[...]) + bvec[...]
    pt = _block_onehot(batch[...])

    @pl.when(jnp.logical_and(p == 0, i == 0))
    def _init():
        cnt_s[...] = jnp.zeros_like(cnt_s)
        sm_s[...] = jnp.zeros_like(sm_s)
        sq_s[...] = jnp.zeros_like(sq_s)
        pool_s[...] = jnp.zeros_like(pool_s)
        cf_s[...] = jnp.zeros_like(cf_s)

    @pl.when(p == 0)
    def _accum():
        ones = jnp.full((BLK, 1), 1.0, jnp.float32)
        cnt_s[...] += _segdot(pt, ones)
        sm_s[...] += _segdot(pt, a)

    @pl.when(jnp.logical_and(p == 1, i == 0))
    def _mean():
        mean_s[...] = sm_s[...] / jnp.maximum(cnt_s[...], 1.0)

    @pl.when(p == 1)
    def _accum2():
        ctr = a - _dot(pt, mean_s[...]) * gms[...]
        sq_s[...] += _segdot(pt, ctr * ctr)

    @pl.when(jnp.logical_and(p == 2, i == 0))
    def _stats():
        c = jnp.maximum(cnt_s[...], 1.0)
        rstd_s[...] = jnp.sqrt(sq_s[...] / c + 1e-5)

    @pl.when(p == 2)
    def _emit():
        ctr = a - _dot(pt, mean_s[...]) * gms[...]
        g = ctr / _dot(pt, rstd_s[...]) * gw[...] + gb[...]
        h = jnp.maximum(g, 0.0)
        pool_s[...] += _segdot(pt, h)
        rowid = lax.broadcasted_iota(jnp.int32, (BLK, G), 0) + i * BLK
        cm = (rowid == cent[...]).astype(jnp.float32)
        cf_s[...] += _segdot(cm, h)

    @pl.when(jnp.logical_and(p == 2, i == NBLK - 1))
    def _mlp():
        c = jnp.maximum(cnt_s[...], 1.0)
        xg = pool_s[...] / c
        xc = jnp.concatenate([xg, cf_s[...]], axis=1)
        m = jnp.maximum(_dotd(xc, wm1[...]) + bm1[...], 0.0)
        out_ref[...] = _dotd(m, wm2[...]) + bm2[...]


_final_call = pl.pallas_call(
    _final_body,
    grid=(3, NBLK),
    in_specs=[pl.BlockSpec((BLK, H), lambda p, i: (i, 0)),
              pl.BlockSpec((BLK, H), lambda p, i: (i, 0)),
              pl.BlockSpec((BLK, H), lambda p, i: (i, 0)),
              pl.BlockSpec((BLK, 1), lambda p, i: (i, 0)),
              pl.BlockSpec((BLK, 1), lambda p, i: (i, 0)),
              pl.BlockSpec((1, G), lambda p, i: (0, 0)),
              pl.BlockSpec((1, H), lambda p, i: (0, 0)),
              pl.BlockSpec((1, H), lambda p, i: (0, 0)),
              pl.BlockSpec((1, H), lambda p, i: (0, 0)),
              pl.BlockSpec((1, H), lambda p, i: (0, 0)),
              pl.BlockSpec((2 * H, H), lambda p, i: (0, 0)),
              pl.BlockSpec((1, H), lambda p, i: (0, 0)),
              pl.BlockSpec((H, 1), lambda p, i: (0, 0)),
              pl.BlockSpec((1, 1), lambda p, i: (0, 0))],
    out_specs=pl.BlockSpec((G, 1), lambda p, i: (0, 0)),
    out_shape=jax.ShapeDtypeStruct((G, 1), jnp.float32),
    scratch_shapes=[pltpu.VMEM((G, 1), jnp.float32),
                    pltpu.VMEM((G, H), jnp.float32),
                    pltpu.VMEM((G, H), jnp.float32),
                    pltpu.VMEM((G, H), jnp.float32),
                    pltpu.VMEM((G, H), jnp.float32),
                    pltpu.VMEM((G, H), jnp.float32),
                    pltpu.VMEM((G, H), jnp.float32)],
)


def kernel(x, edge_index, batch, center_idx, params):
    p = params
    src = edge_index[0].astype(jnp.int32)
    dst = edge_index[1].astype(jnp.int32)
    pad = EPAD - E
    srcr = jnp.concatenate([src, jnp.zeros((pad,), jnp.int32)]).reshape(NW, NB_W, B)
    dstr = jnp.concatenate([dst, jnp.full((pad,), N, jnp.int32)]).reshape(NW, NB_W, B)

    zeros8 = jnp.zeros((NPAD, DW), jnp.float32)
    ones8 = jnp.ones((B, DW), jnp.float32)
    zeros_h = jnp.zeros((NPAD, H), jnp.float32)

    degp = _deg_call()(dstr, zeros8, ones8)
    d0 = degp[0, :N, 0:1]
    d1 = degp[1, :N, 0:1]

    batch_col = batch.astype(jnp.int32)[:, None]
    cent_row = center_idx.astype(jnp.int32)[None, :]

    def vec(v):
        return v[None, :]

    dinv, hs = _pre_call(x, p['W0'], d0, d1)

    pad_rows = jnp.zeros((NPAD - N, H), jnp.float32)
    h_res = None
    for i in range(4):
        hs_pad = jnp.concatenate([hs, pad_rows], axis=0)
        accp = _edge_call()(hs_pad, srcr, dstr, zeros_h)
        acc0 = accp[0, :N, :]
        acc1 = accp[1, :N, :]
        if i < 3:
            args = (acc0, acc1, hs, dinv, batch_col, vec(p['b%d' % i]),
                    vec(p['gn%d_w' % i]), vec(p['gn%d_b' % i]),
                    vec(p['gn%d_ms' % i]), p['W%d' % (i + 1)])
            if i == 0:
                h_res, hs = _mid_nores(*args)
            else:
                h_res, hs = _mid_res(*args, h_res)
        else:
            out = _final_call(acc0, acc1, hs, dinv, batch_col, cent_row,
                              vec(p['b3']), vec(p['gn3_w']), vec(p['gn3_b']),
                              vec(p['gn3_ms']), p['Wm1'], vec(p['bm1']),
                              p['Wm2'], vec(p['bm2']))
    return out[:, 0]


# speedup vs baseline: 21.8682x; 1.1796x over previous
import functools

import jax
import jax.numpy as jnp
from jax import lax
from jax.experimental import pallas as pl
from jax.experimental.pallas import tpu as pltpu
from jax.experimental.pallas import tpu_sc as plsc

N = 10000
E = 320000
DF = 128
H = 64
G = 16

NC = 2
NS = 16
NW = NC * NS
B = 128
NB_W = -(-E // (NW * B))
EPAD = NW * B * NB_W
NPAD = 10240
RPT = NPAD // NS
DW = 8


def _deg_body(dstr, zeros8, ones8, out, didx, ones_v, deg_sh):
    cid = lax.axis_index("c")
    sid = lax.axis_index("s")
    wid = cid * NS + sid
    pltpu.sync_copy(zeros8.at[pl.ds(sid * RPT, RPT)],
                    deg_sh.at[pl.ds(sid * RPT, RPT)])
    pltpu.sync_copy(ones8, ones_v)
    pltpu.sync_copy(dstr.at[wid], didx)
    plsc.subcore_barrier()
    for j in range(NB_W):
        pltpu.sync_copy(ones_v, deg_sh.at[didx.at[j]], add=True)
    plsc.subcore_barrier()
    pltpu.sync_copy(deg_sh.at[pl.ds(sid * RPT, RPT)],
                    out.at[cid, pl.ds(sid * RPT, RPT)])


@functools.cache
def _deg_call():
    mesh = plsc.VectorSubcoreMesh(
        core_axis_name="c", subcore_axis_name="s",
        num_cores=NC, num_subcores=NS)
    return pl.kernel(
        _deg_body,
        out_type=jax.ShapeDtypeStruct((NC, NPAD, DW), jnp.float32),
        mesh=mesh,
        scratch_types=[
            pltpu.VMEM((NB_W, B), jnp.int32),
            pltpu.VMEM((B, DW), jnp.float32),
            pltpu.VMEM_SHARED((NPAD, DW), jnp.float32),
        ],
        compiler_params=pltpu.CompilerParams(use_tc_tiling_on_sc=False),
    )


def _edge_body(hs, srcr, dstr, zeros_h, out,
               sidx, didx, rows0, rows1, rows2, acc_sh, hs_sh, s0, s1, s2):
    cid = lax.axis_index("c")
    sid = lax.axis_index("s")
    wid = cid * NS + sid
    pltpu.sync_copy(zeros_h.at[pl.ds(sid * RPT, RPT)],
                    acc_sh.at[pl.ds(sid * RPT, RPT)])
    pltpu.sync_copy(hs.at[pl.ds(sid * RPT, RPT)],
                    hs_sh.at[pl.ds(sid * RPT, RPT)])
    pltpu.sync_copy(srcr.at[wid], sidx)
    pltpu.sync_copy(dstr.at[wid], didx)
    plsc.subcore_barrier()
    rows = (rows0, rows1, rows2)
    sems = (s0, s1, s2)
    descs = [None] * NB_W
    for j in range(min(2, NB_W)):
        descs[j] = pltpu.async_copy(hs_sh.at[sidx.at[j]], rows[j % 3], sems[j % 3])
    for j in range(NB_W):
        if j + 2 < NB_W:
            descs[j + 2] = pltpu.async_copy(
                hs_sh.at[sidx.at[j + 2]], rows[(j + 2) % 3], sems[(j + 2) % 3])
        descs[j].wait()
        pltpu.sync_copy(rows[j % 3], acc_sh.at[didx.at[j]], add=True)
    plsc.subcore_barrier()
    pltpu.sync_copy(acc_sh.at[pl.ds(sid * RPT, RPT)],
                    out.at[cid, pl.ds(sid * RPT, RPT)])


@functools.cache
def _edge_call():
    mesh = plsc.VectorSubcoreMesh(
        core_axis_name="c", subcore_axis_name="s",
        num_cores=NC, num_subcores=NS)
    return pl.kernel(
        _edge_body,
        out_type=jax.ShapeDtypeStruct((NC, NPAD, H), jnp.float32),
        mesh=mesh,
        scratch_types=[
            pltpu.VMEM((NB_W, B), jnp.int32),
            pltpu.VMEM((NB_W, B), jnp.int32),
            pltpu.VMEM((B, H), jnp.float32),
            pltpu.VMEM((B, H), jnp.float32),
            pltpu.VMEM((B, H), jnp.float32),
            pltpu.VMEM_SHARED((NPAD, H), jnp.float32),
            pltpu.VMEM_SHARED((NPAD, H), jnp.float32),
            pltpu.SemaphoreType.DMA,
            pltpu.SemaphoreType.DMA,
            pltpu.SemaphoreType.DMA,
        ],
        compiler_params=pltpu.CompilerParams(use_tc_tiling_on_sc=False),
    )


BLK = 2000
NBLK = N // BLK


def _dot(a, b):
    return jnp.dot(a, b, preferred_element_type=jnp.float32,
                   precision=lax.Precision.HIGHEST)


def _dotd(a, b):
    return jnp.dot(a, b, preferred_element_type=jnp.float32)


def _segdot(pt, v):
    return lax.dot_general(pt, v, (((0,), (0,)), ((), ())),
                           preferred_element_type=jnp.float32,
                           precision=lax.Precision.HIGHEST)


def _pre_body(x_ref, w0_ref, d0_ref, d1_ref, dinv_ref, hs0_ref):
    d = d0_ref[...] + d1_ref[...] + 1.0
    dinv = lax.rsqrt(jnp.maximum(d, 1.0))
    dinv_ref[...] = dinv
    hs0_ref[...] = _dotd(x_ref[...], w0_ref[...]) * dinv


_pre_call = pl.pallas_call(
    _pre_body,
    grid=(NBLK,),
    in_specs=[pl.BlockSpec((BLK, DF), lambda i: (i, 0)),
              pl.BlockSpec((DF, H), lambda i: (0, 0)),
              pl.BlockSpec((BLK, 1), lambda i: (i, 0)),
              pl.BlockSpec((BLK, 1), lambda i: (i, 0))],
    out_specs=[pl.BlockSpec((BLK, 1), lambda i: (i, 0)),
               pl.BlockSpec((BLK, H), lambda i: (i, 0))],
    out_shape=[jax.ShapeDtypeStruct((N, 1), jnp.float32),
               jax.ShapeDtypeStruct((N, H), jnp.float32)],
)


def _block_onehot(batch_blk):
    return (batch_blk == lax.broadcasted_iota(jnp.int32, (BLK, G), 1)
            ).astype(jnp.float32)


def _layer_body(with_res, with_mm, *refs):
    (acc0, acc1, hs_prev, dinv, batch, bvec, gw, gb, gms) = refs[:9]
    k = 9
    wn = refs[k] if with_mm else None
    k += int(with_mm)
    hres = refs[k] if with_res else None
    k += int(with_res)
    h_new_ref = refs[k]
    k += 1
    hs_next_ref = refs[k] if with_mm else None
    k += int(with_mm)
    cnt_s, sm_s, sq_s, mean_s, rstd_s = refs[k:k + 5]

    p = pl.program_id(0)
    i = pl.program_id(1)
    a = ((acc0[...] + acc1[...] + hs_prev[...]) * dinv[...]) + bvec[...]
    pt = _block_onehot(batch[...])

    @pl.when(jnp.logical_and(p == 0, i == 0))
    def _init():
        cnt_s[...] = jnp.zeros_like(cnt_s)
        sm_s[...] = jnp.zeros_like(sm_s)
        sq_s[...] = jnp.zeros_like(sq_s)

    @pl.when(p == 0)
    def _accum():
        ones = jnp.full((BLK, 1), 1.0, jnp.float32)
        cnt_s[...] += _segdot(pt, ones)
        sm_s[...] += _segdot(pt, a)

    @pl.when(jnp.logical_and(p == 1, i == 0))
    def _mean():
        mean_s[...] = sm_s[...] / jnp.maximum(cnt_s[...], 1.0)

    @pl.when(p == 1)
    def _accum2():
        ctr = a - _dot(pt, mean_s[...]) * gms[...]
        sq_s[...] += _segdot(pt, ctr * ctr)

    @pl.when(jnp.logical_and(p == 2, i == 0))
    def _stats():
        c = jnp.maximum(cnt_s[...], 1.0)
        rstd_s[...] = jnp.sqrt(sq_s[...] / c + 1e-5)

    @pl.when(p == 2)
    def _emit():
        ctr = a - _dot(pt, mean_s[...]) * gms[...]
        g = ctr / _dot(pt, rstd_s[...]) * gw[...] + gb[...]
        g = jnp.maximum(g, 0.0)
        h_new = (hres[...] + g) if with_res else g
        h_new_ref[...] = h_new
        if with_mm:
            hs_next_ref[...] = _dotd(h_new, wn[...]) * dinv[...]


def _make_layer(with_res):
    row = lambda w: pl.BlockSpec((BLK, w), lambda p, i: (i, 0))
    const = lambda r, c: pl.BlockSpec((r, c), lambda p, i: (0, 0))
    in_specs = [row(H), row(H), row(H), row(1), row(1),
                const(1, H), const(1, H), const(1, H), const(1, H),
                const(H, H)]
    if with_res:
        in_specs.append(row(H))
    return pl.pallas_call(
        functools.partial(_layer_body, with_res, True),
        grid=(3, NBLK),
        in_specs=in_specs,
        out_specs=[row(H), row(H)],
        out_shape=[jax.ShapeDtypeStruct((N, H), jnp.float32),
                   jax.ShapeDtypeStruct((N, H), jnp.float32)],
        scratch_shapes=[pltpu.VMEM((G, 1), jnp.float32),
                        pltpu.VMEM((G, H), jnp.float32),
                        pltpu.VMEM((G, H), jnp.float32),
                        pltpu.VMEM((G, H), jnp.float32),
                        pltpu.VMEM((G, H), jnp.float32)],
    )


_mid_nores = _make_layer(False)
_mid_res = _make_layer(True)


def _final_body(acc0, acc1, hs3, dinv, batch, cent, bvec, gw, gb, gms,
                wm1, bm1, wm2, bm2, out_ref,
                cnt_s, sm_s, sq_s, mean_s, rstd_s, pool_s, cf_s):
    p = pl.program_id(0)
    i = pl.program_id(1)
    a = ((acc0[...] + acc1[...] + hs3[...]) * dinv[...]) + bvec[...]
    pt = _block_onehot(batch[...])

    @pl.when(jnp.logical_and(p == 0, i == 0))
    def _init():
        cnt_s[...] = jnp.zeros_like(cnt_s)
        sm_s[...] = jnp.zeros_like(sm_s)
        sq_s[...] = jnp.zeros_like(sq_s)
        pool_s[...] = jnp.zeros_like(pool_s)
        cf_s[...] = jnp.zeros_like(cf_s)

    @pl.when(p == 0)
    def _accum():
        ones = jnp.full((BLK, 1), 1.0, jnp.float32)
        cnt_s[...] += _segdot(pt, ones)
        sm_s[...] += _segdot(pt, a)

    @pl.when(jnp.logical_and(p == 1, i == 0))
    def _mean():
        mean_s[...] = sm_s[...] / jnp.maximum(cnt_s[...], 1.0)

    @pl.when(p == 1)
    def _accum2():
        ctr = a - _dot(pt, mean_s[...]) * gms[...]
        sq_s[...] += _segdot(pt, ctr * ctr)

    @pl.when(jnp.logical_and(p == 2, i == 0))
    def _stats():
        c = jnp.maximum(cnt_s[...], 1.0)
        rstd_s[...] = jnp.sqrt(sq_s[...] / c + 1e-5)

    @pl.when(p == 2)
    def _emit():
        ctr = a - _dot(pt, mean_s[...]) * gms[...]
        g = ctr / _dot(pt, rstd_s[...]) * gw[...] + gb[...]
        h = jnp.maximum(g, 0.0)
        pool_s[...] += _segdot(pt, h)
        rowid = lax.broadcasted_iota(jnp.int32, (BLK, G), 0) + i * BLK
        cm = (rowid == cent[...]).astype(jnp.float32)
        cf_s[...] += _segdot(cm, h)

    @pl.when(jnp.logical_and(p == 2, i == NBLK - 1))
    def _mlp():
        c = jnp.maximum(cnt_s[...], 1.0)
        xg = pool_s[...] / c
        xc = jnp.concatenate([xg, cf_s[...]], axis=1)
        m = jnp.maximum(_dotd(xc, wm1[...]) + bm1[...], 0.0)
        out_ref[...] = _dotd(m, wm2[...]) + bm2[...]


_final_call = pl.pallas_call(
    _final_body,
    grid=(3, NBLK),
    in_specs=[pl.BlockSpec((BLK, H), lambda p, i: (i, 0)),
              pl.BlockSpec((BLK, H), lambda p, i: (i, 0)),
              pl.BlockSpec((BLK, H), lambda p, i: (i, 0)),
              pl.BlockSpec((BLK, 1), lambda p, i: (i, 0)),
              pl.BlockSpec((BLK, 1), lambda p, i: (i, 0)),
              pl.BlockSpec((1, G), lambda p, i: (0, 0)),
              pl.BlockSpec((1, H), lambda p, i: (0, 0)),
              pl.BlockSpec((1, H), lambda p, i: (0, 0)),
              pl.BlockSpec((1, H), lambda p, i: (0, 0)),
              pl.BlockSpec((1, H), lambda p, i: (0, 0)),
              pl.BlockSpec((2 * H, H), lambda p, i: (0, 0)),
              pl.BlockSpec((1, H), lambda p, i: (0, 0)),
              pl.BlockSpec((H, 1), lambda p, i: (0, 0)),
              pl.BlockSpec((1, 1), lambda p, i: (0, 0))],
    out_specs=pl.BlockSpec((G, 1), lambda p, i: (0, 0)),
    out_shape=jax.ShapeDtypeStruct((G, 1), jnp.float32),
    scratch_shapes=[pltpu.VMEM((G, 1), jnp.float32),
                    pltpu.VMEM((G, H), jnp.float32),
                    pltpu.VMEM((G, H), jnp.float32),
                    pltpu.VMEM((G, H), jnp.float32),
                    pltpu.VMEM((G, H), jnp.float32),
                    pltpu.VMEM((G, H), jnp.float32),
                    pltpu.VMEM((G, H), jnp.float32)],
)


def kernel(x, edge_index, batch, center_idx, params):
    p = params
    src = edge_index[0].astype(jnp.int32)
    dst = edge_index[1].astype(jnp.int32)
    pad = EPAD - E
    srcr = jnp.concatenate([src, jnp.zeros((pad,), jnp.int32)]).reshape(NW, NB_W, B)
    dstr = jnp.concatenate([dst, jnp.full((pad,), N, jnp.int32)]).reshape(NW, NB_W, B)

    zeros8 = jnp.zeros((NPAD, DW), jnp.float32)
    ones8 = jnp.ones((B, DW), jnp.float32)
    zeros_h = jnp.zeros((NPAD, H), jnp.float32)

    degp = _deg_call()(dstr, zeros8, ones8)
    d0 = degp[0, :N, 0:1]
    d1 = degp[1, :N, 0:1]

    batch_col = batch.astype(jnp.int32)[:, None]
    cent_row = center_idx.astype(jnp.int32)[None, :]

    def vec(v):
        return v[None, :]

    dinv, hs = _pre_call(x, p['W0'], d0, d1)

    pad_rows = jnp.zeros((NPAD - N, H), jnp.float32)
    h_res = None
    for i in range(4):
        hs_pad = jnp.concatenate([hs, pad_rows], axis=0)
        accp = _edge_call()(hs_pad, srcr, dstr, zeros_h)
        acc0 = accp[0, :N, :]
        acc1 = accp[1, :N, :]
        if i < 3:
            args = (acc0, acc1, hs, dinv, batch_col, vec(p['b%d' % i]),
                    vec(p['gn%d_w' % i]), vec(p['gn%d_b' % i]),
                    vec(p['gn%d_ms' % i]), p['W%d' % (i + 1)])
            if i == 0:
                h_res, hs = _mid_nores(*args)
            else:
                h_res, hs = _mid_res(*args, h_res)
        else:
            out = _final_call(acc0, acc1, hs, dinv, batch_col, cent_row,
                              vec(p['b3']), vec(p['gn3_w']), vec(p['gn3_b']),
                              vec(p['gn3_ms']), p['Wm1'], vec(p['bm1']),
                              p['Wm2'], vec(p['bm2']))
    return out[:, 0]
